# baseline (speedup 1.0000x reference)
import functools

import jax
import jax.numpy as jnp
from jax import lax
from jax.experimental import pallas as pl
from jax.experimental.pallas import tpu as pltpu
from jax.experimental.pallas import tpu_sc as plsc

_NUM_CORES = 2
_NUM_SUBCORES = 16
_NUM_WORKERS = _NUM_CORES * _NUM_SUBCORES
_LANES = 16
_CHUNK = 128


def _sc_body(batch, vocab, n_ids, scores_hbm, ids_hbm, out_hbm,
             ids_v, row_v, vals_v, idx_v, gsem, osem):
    rows_per_worker = batch // _NUM_WORKERS
    n_chunks = n_ids // _CHUNK

    cid = lax.axis_index("c")
    sid = lax.axis_index("s")
    wid = sid * _NUM_CORES + cid
    row0 = wid * rows_per_worker

    pltpu.sync_copy(ids_hbm, ids_v)

    def start_gather(r, buf):
        grow = row0 + r

        def mk_idx(j, carry):
            v = ids_v[pl.ds(j * _LANES, _LANES)] + grow * vocab
            idx_v[buf * n_chunks + j * _LANES // _CHUNK,
                  pl.ds((j * _LANES) % _CHUNK, _LANES)] = v
            return carry

        lax.fori_loop(0, n_ids // _LANES, mk_idx, 0)
        descs = []
        for k in range(n_chunks):
            d = pltpu.async_copy(
                scores_hbm.at[idx_v.at[buf * n_chunks + k]],
                vals_v.at[buf * n_chunks + k],
                gsem,
            )
            descs.append(d)
        return descs

    gdescs = start_gather(0, 0)

    neg_inf = jnp.full((_LANES,), -jnp.inf, dtype=jnp.float32)

    def fill(i, carry):
        row_v[pl.ds(i * _LANES, _LANES)] = neg_inf
        return carry

    lax.fori_loop(0, vocab // _LANES, fill, 0)

    out_desc = None
    for r in range(rows_per_worker):
        buf = r % 2
        if r + 1 < rows_per_worker:
            next_descs = start_gather(r + 1, 1 - buf)
        else:
            next_descs = None
        for d in gdescs:
            d.wait()
        if out_desc is not None:
            out_desc.wait()

        def scatter(i, carry):
            val = vals_v[buf * n_chunks + i * _LANES // _CHUNK,
                         pl.ds((i * _LANES) % _CHUNK, _LANES)]
            ivec = ids_v[pl.ds(i * _LANES, _LANES)]
            plsc.store_scatter(row_v, [ivec], val)
            return carry

        lax.fori_loop(0, n_ids // _LANES, scatter, 0)

        out_desc = pltpu.async_copy(row_v, out_hbm.at[row0 + r], osem)
        gdescs = next_descs
    out_desc.wait()


def kernel(input_ids, scores, allowed_token_ids):
    del input_ids
    batch, vocab = scores.shape
    n_ids = allowed_token_ids.shape[0]
    ids = allowed_token_ids.astype(jnp.int32)
    scores_flat = scores.reshape(batch * vocab)

    mesh = plsc.VectorSubcoreMesh(core_axis_name="c", subcore_axis_name="s")
    n_chunks = n_ids // _CHUNK
    out = pl.kernel(
        functools.partial(_sc_body, batch, vocab, n_ids),
        out_type=jax.ShapeDtypeStruct((batch, vocab), jnp.float32),
        mesh=mesh,
        scratch_types=[
            pltpu.VMEM((n_ids,), jnp.int32),
            pltpu.VMEM((vocab,), jnp.float32),
            pltpu.VMEM((2 * n_chunks, _CHUNK), jnp.float32),
            pltpu.VMEM((2 * n_chunks, _CHUNK), jnp.int32),
            pltpu.SemaphoreType.DMA,
            pltpu.SemaphoreType.DMA,
        ],
        compiler_params=pltpu.CompilerParams(needs_layout_passes=False),
        name="sc_sparse_logits_mask",
    )(scores_flat, ids)
    return out


# speedup vs baseline: 4.1246x; 4.1246x over previous
import functools

import jax
import jax.numpy as jnp
from jax import lax
from jax.experimental import pallas as pl
from jax.experimental.pallas import tpu as pltpu
from jax.experimental.pallas import tpu_sc as plsc

_NUM_CORES = 2
_NUM_SUBCORES = 16
_NUM_WORKERS = _NUM_CORES * _NUM_SUBCORES
_LANES = 16
_CHUNK = 128


def _sc_body(batch, vocab, n_ids, scores_hbm, ids_hbm, out_hbm,
             ids_v, neg_v, vals_v, loc_v, idx2_v, idxf_v, isem, fsem, gsem):
    rows = vocab // _NUM_WORKERS
    n_fill = idxf_v.shape[0]
    n_vecs = n_ids // _LANES
    loc_vecs = loc_v.shape[0] // _LANES

    cid = lax.axis_index("c")
    sid = lax.axis_index("s")
    wid = sid * _NUM_CORES + cid
    lo = wid * rows
    hi = lo + rows

    ids_cp = pltpu.async_copy(ids_hbm, ids_v, isem)

    c16 = jnp.arange(_LANES, dtype=jnp.int32)

    neg_inf = jnp.full((_LANES,), -jnp.inf, dtype=jnp.float32)

    def neg_init(i, carry):
        neg_v[i // 8, pl.ds((i % 8) * _LANES, _LANES)] = neg_inf
        return carry

    lax.fori_loop(0, _CHUNK * batch // _LANES, neg_init, 0)

    def fill_idx(c, carry):
        row = jnp.minimum(lo + c * _LANES + c16, hi - 1)
        idxf_v[c // 8, pl.ds((c % 8) * _LANES, _LANES)] = row
        return carry

    lax.fori_loop(0, n_fill * _CHUNK // _LANES, fill_idx, 0)

    fdescs = [
        pltpu.async_copy(neg_v, out_hbm.at[idxf_v.at[k]], fsem)
        for k in range(n_fill)
    ]

    ids_cp.wait()

    def compact(i, k):
        v = ids_v[pl.ds(i * _LANES, _LANES)]
        m = (v >= lo) & (v < hi)
        plsc.store_compressed(loc_v.at[pl.ds(k, _LANES)], v, mask=m)
        return k + jnp.sum(m.astype(jnp.int32))

    n_local = lax.fori_loop(0, n_vecs, compact, 0)

    for d in fdescs:
        d.wait()

    @pl.when(n_local > 0)
    def _():
        first = loc_v[pl.ds(0, _LANES)]
        pad_id = jnp.min(jnp.where(c16 < n_local, first, jnp.int32(2**30)))

        def pad(c, carry):
            cur = loc_v[pl.ds(c * _LANES, _LANES)]
            keep = (c * _LANES + c16) < n_local
            loc_v[pl.ds(c * _LANES, _LANES)] = jnp.where(
                keep, cur, jnp.zeros_like(cur) + pad_id)
            return carry

        lax.fori_loop(0, loc_vecs, pad, 0)

        def mirror(c, carry):
            idx2_v[c // 8, pl.ds((c % 8) * _LANES, _LANES)] = (
                loc_v[pl.ds(c * _LANES, _LANES)])
            return carry

        lax.fori_loop(0, loc_vecs, mirror, 0)

        n_chunks = (n_local + _CHUNK - 1) // _CHUNK

        def move(c, carry):
            pltpu.async_copy(scores_hbm.at[idx2_v.at[c]], vals_v, gsem).wait()
            pltpu.async_copy(vals_v, out_hbm.at[idx2_v.at[c]], gsem).wait()
            return carry

        lax.fori_loop(0, n_chunks, move, 0)


def kernel(input_ids, scores, allowed_token_ids):
    del input_ids
    batch, vocab = scores.shape
    n_ids = allowed_token_ids.shape[0]
    ids = allowed_token_ids.astype(jnp.int32)
    scores_t = scores.T

    rows = vocab // _NUM_WORKERS
    n_fill = -(-rows // _CHUNK)
    loc_cap = n_ids + _CHUNK
    mesh = plsc.VectorSubcoreMesh(core_axis_name="c", subcore_axis_name="s")
    out_t = pl.kernel(
        functools.partial(_sc_body, batch, vocab, n_ids),
        out_type=jax.ShapeDtypeStruct((vocab, batch), jnp.float32),
        mesh=mesh,
        scratch_types=[
            pltpu.VMEM((n_ids,), jnp.int32),
            pltpu.VMEM((_CHUNK, batch), jnp.float32),
            pltpu.VMEM((_CHUNK, batch), jnp.float32),
            pltpu.VMEM((loc_cap,), jnp.int32),
            pltpu.VMEM((loc_cap // _CHUNK, _CHUNK), jnp.int32),
            pltpu.VMEM((n_fill, _CHUNK), jnp.int32),
            pltpu.SemaphoreType.DMA,
            pltpu.SemaphoreType.DMA,
            pltpu.SemaphoreType.DMA,
        ],
        compiler_params=pltpu.CompilerParams(needs_layout_passes=False),
        name="sc_sparse_logits_mask",
    )(scores_t, ids)
    return out_t.T
